# baseline (speedup 1.0000x reference)
import jax
import jax.numpy as jnp
from jax import lax
from jax.experimental import pallas as pl
from jax.experimental.pallas import tpu as pltpu
from jax.experimental.pallas import tpu_sc as plsc

N_NODES = 10000
D = 128
W = D // 2
E = 320000
NC = 2
NS = 16
NW = NC * NS
E_T = E // NW
C = 80
N_CHUNKS = E_T // C
UNROLL = 16


def _sc_body(zp_hbm, src_hbm, dst_hbm, out_hbm,
             sidx, didx, srows, drows, outc, sems):
    wid = lax.axis_index("c") * NS + lax.axis_index("s")
    tile_base = wid * E_T

    def start(ic, b):
        base = tile_base + ic * C
        pltpu.sync_copy(src_hbm.at[pl.ds(base, C)], sidx.at[b])
        pltpu.sync_copy(dst_hbm.at[pl.ds(base, C)], didx.at[b])
        pltpu.async_copy(zp_hbm.at[sidx.at[b]], srows.at[b], sems.at[b])
        pltpu.async_copy(zp_hbm.at[didx.at[b]], drows.at[b], sems.at[b])

    def wait(b):
        pltpu.make_async_copy(zp_hbm.at[sidx.at[b]], srows.at[b], sems.at[b]).wait()
        pltpu.make_async_copy(zp_hbm.at[didx.at[b]], drows.at[b], sems.at[b]).wait()

    def compute(ic, b):
        base = tile_base + ic * C

        def g_body(g, _):
            rows = lax.iota(jnp.int32, 16) + g * 16
            skew = lax.iota(jnp.int32, 16)

            def d_body(dblk, accs):
                acc0, acc1 = accs
                for j in range(UNROLL):
                    col = (skew + (dblk * UNROLL + j)) & (W - 1)
                    a = plsc.load_gather(srows.at[b], [rows, col])
                    bb = plsc.load_gather(drows.at[b], [rows, col])
                    ab = plsc.bitcast(a, jnp.bfloat16)
                    bbb = plsc.bitcast(bb, jnp.bfloat16)
                    p = plsc.bitcast(ab * bbb, jnp.int32)
                    plo = plsc.bitcast(p << 16, jnp.float32)
                    phi = plsc.bitcast(p & jnp.int32(-65536), jnp.float32)
                    acc0 = acc0 + plo
                    acc1 = acc1 + phi
                return acc0, acc1

            acc0, acc1 = lax.fori_loop(
                0, W // UNROLL, d_body,
                (jnp.zeros((16,), jnp.float32), jnp.zeros((16,), jnp.float32)))
            outc[pl.ds(g * 16, 16)] = acc0 + acc1
            return _

        lax.fori_loop(0, C // 16, g_body, 0)
        pltpu.sync_copy(outc, out_hbm.at[pl.ds(base, C)])

    start(0, 0)
    start(1, 1)

    def pair_body(i, _):
        for b in range(2):
            ic = i * 2 + b
            wait(b)
            compute(ic, b)

            @pl.when(ic + 2 < N_CHUNKS)
            def _start_next():
                start(ic + 2, b)

        return _

    lax.fori_loop(0, N_CHUNKS // 2, pair_body, 0)
    wait(0)
    compute(N_CHUNKS - 1, 0)


@jax.jit
def kernel(z, edge_index):
    src = edge_index[0].astype(jnp.int32)
    dst = edge_index[1].astype(jnp.int32)
    zp = lax.bitcast_convert_type(
        z.astype(jnp.bfloat16).reshape(N_NODES, W, 2), jnp.int32)
    zp = jnp.pad(zp, ((0, 0), (0, D - W)))
    mesh = plsc.VectorSubcoreMesh(core_axis_name="c", subcore_axis_name="s")
    f = pl.kernel(
        _sc_body,
        out_type=jax.ShapeDtypeStruct((E,), jnp.float32),
        mesh=mesh,
        scratch_types=[
            pltpu.VMEM((2, C), jnp.int32),
            pltpu.VMEM((2, C), jnp.int32),
            pltpu.VMEM((2, C, D), jnp.int32),
            pltpu.VMEM((2, C, D), jnp.int32),
            pltpu.VMEM((C,), jnp.float32),
            pltpu.SemaphoreType.DMA((2,)),
        ],
        compiler_params=pltpu.CompilerParams(needs_layout_passes=False),
    )
    return f(zp, src, dst)
